# Initial kernel scaffold; baseline (speedup 1.0000x reference)
import functools

import jax
import jax.numpy as jnp
from jax import lax
from jax.experimental import pallas as pl
from jax.experimental.pallas import tpu as pltpu
from jax.experimental.pallas import tpu_sc as plsc

EMB = 64
CHUNK = 128


@functools.partial(jax.jit, static_argnames=("total",))
def _sc_embedding_gather(table, idx_flat, total):
    info = plsc.get_sparse_core_info()
    num_workers = info.num_cores * info.num_subcores
    per_worker = total // num_workers
    n_chunks = per_worker // CHUNK
    mesh = plsc.VectorSubcoreMesh(core_axis_name="c", subcore_axis_name="s")

    @functools.partial(
        pl.kernel,
        mesh=mesh,
        out_type=jax.ShapeDtypeStruct((total, EMB), jnp.float32),
        scratch_types=[
            pltpu.VMEM((CHUNK,), jnp.int32),
            pltpu.VMEM((CHUNK, EMB), jnp.float32),
            pltpu.SemaphoreType.DMA,
        ],
    )
    def k(table_hbm, idx_hbm, out_hbm, idx_v, rows_v, sem):
        wid = lax.axis_index("s") * info.num_cores + lax.axis_index("c")
        base = wid * per_worker

        def body(g, carry):
            off = base + g * CHUNK
            pltpu.sync_copy(idx_hbm.at[pl.ds(off, CHUNK)], idx_v)
            pltpu.async_copy(table_hbm.at[idx_v], rows_v, sem).wait()
            pltpu.sync_copy(rows_v, out_hbm.at[pl.ds(off, CHUNK)])
            return carry

        lax.fori_loop(0, n_chunks, body, 0)

    return k(table, idx_flat)


def kernel(atomic_numbers, embedding_table):
    total = atomic_numbers.size
    idx_flat = atomic_numbers.reshape(total).astype(jnp.int32)
    out = _sc_embedding_gather(embedding_table, idx_flat, total)
    return out.reshape(atomic_numbers.shape + (EMB,))


# speedup vs baseline: 3.3079x; 3.3079x over previous
import functools

import jax
import jax.numpy as jnp
from jax import lax
from jax.experimental import pallas as pl
from jax.experimental.pallas import tpu as pltpu
from jax.experimental.pallas import tpu_sc as plsc

EMB = 64
VOCAB_ROWS = 129
PAIRS = 128
LANES = 16


@functools.partial(jax.jit, static_argnames=("total",))
def _sc_embedding_gather(pair_table, idx_flat, total):
    info = plsc.get_sparse_core_info()
    num_workers = info.num_cores * info.num_subcores
    pairs_total = total // 2
    per_worker = pairs_total // num_workers
    n_chunks = per_worker // PAIRS
    mesh = plsc.VectorSubcoreMesh(core_axis_name="c", subcore_axis_name="s")

    @functools.partial(
        pl.kernel,
        mesh=mesh,
        out_type=jax.ShapeDtypeStruct((pairs_total, 2 * EMB), jnp.float32),
        scratch_types=[
            pltpu.VMEM((PAIRS,), jnp.int32),
            pltpu.VMEM((PAIRS, 2 * EMB), jnp.float32),
            pltpu.SemaphoreType.DMA,
        ],
    )
    def k(pt_hbm, pidx_hbm, out_hbm, pidx_v, rows_v, sem):
        wid = lax.axis_index("s") * info.num_cores + lax.axis_index("c")
        base = wid * per_worker

        def body(g, carry):
            off = base + g * PAIRS
            pltpu.sync_copy(pidx_hbm.at[pl.ds(off, PAIRS)], pidx_v)
            pltpu.async_copy(pt_hbm.at[pidx_v], rows_v, sem).wait()
            pltpu.sync_copy(rows_v, out_hbm.at[pl.ds(off, PAIRS)])
            return carry

        lax.fori_loop(0, n_chunks, body, 0)

    return k(pair_table, idx_flat)


def kernel(atomic_numbers, embedding_table):
    total = atomic_numbers.size
    idx = atomic_numbers.reshape(total // 2, 2).astype(jnp.int32)
    idx_flat = idx[:, 0] * VOCAB_ROWS + idx[:, 1]
    v = embedding_table.shape[0]
    pair_table = jnp.concatenate(
        [
            jnp.broadcast_to(embedding_table[:, None, :], (v, v, EMB)),
            jnp.broadcast_to(embedding_table[None, :, :], (v, v, EMB)),
        ],
        axis=-1,
    ).reshape(v * v, 2 * EMB)
    out = _sc_embedding_gather(pair_table, idx_flat, total)
    return out.reshape(atomic_numbers.shape + (EMB,))


# speedup vs baseline: 3.7802x; 1.1428x over previous
import functools

import jax
import jax.numpy as jnp
from jax import lax
from jax.experimental import pallas as pl
from jax.experimental.pallas import tpu as pltpu
from jax.experimental.pallas import tpu_sc as plsc

EMB = 64
VOCAB_ROWS = 129
CHUNK = 400
SUBS = ((0, 128), (128, 128), (256, 128), (384, 16))


@functools.partial(jax.jit, static_argnames=("total",))
def _sc_embedding_gather(pair_table, pidx, total):
    info = plsc.get_sparse_core_info()
    num_workers = info.num_cores * info.num_subcores
    pairs_total = total // 2
    per_worker = pairs_total // num_workers
    n_chunks = per_worker // CHUNK
    half_t = n_chunks // 2
    mesh = plsc.VectorSubcoreMesh(core_axis_name="c", subcore_axis_name="s")

    @functools.partial(
        pl.kernel,
        mesh=mesh,
        out_type=jax.ShapeDtypeStruct((pairs_total, 2 * EMB), jnp.float32),
        scratch_types=[
            pltpu.VMEM((CHUNK,), jnp.int32),
            pltpu.VMEM((CHUNK,), jnp.int32),
            pltpu.VMEM((CHUNK, 2 * EMB), jnp.float32),
            pltpu.VMEM((CHUNK, 2 * EMB), jnp.float32),
            pltpu.SemaphoreType.DMA,
            pltpu.SemaphoreType.DMA,
            pltpu.SemaphoreType.DMA,
            pltpu.SemaphoreType.DMA,
        ],
    )
    def k(pt_hbm, pidx_hbm, out_hbm, pidx0, pidx1, rows0, rows1,
          gsem0, gsem1, osem0, osem1):
        wid = lax.axis_index("s") * info.num_cores + lax.axis_index("c")
        base = wid * per_worker

        def fire_gather(pidx_v, rows_v, gsem):
            for off, sz in SUBS:
                pltpu.async_copy(
                    pt_hbm.at[pidx_v.at[pl.ds(off, sz)]],
                    rows_v.at[pl.ds(off, sz)],
                    gsem,
                )

        def drain_gather(rows_v, gsem):
            pltpu.make_async_copy(out_hbm.at[pl.ds(0, CHUNK)], rows_v, gsem).wait()

        def drain_out(rows_v, osem):
            pltpu.make_async_copy(rows_v, out_hbm.at[pl.ds(0, CHUNK)], osem).wait()

        def load_idx(g, pidx_v):
            pltpu.sync_copy(pidx_hbm.at[pl.ds(base + g * CHUNK, CHUNK)], pidx_v)

        def fire_out(g, rows_v, osem):
            pltpu.async_copy(rows_v, out_hbm.at[pl.ds(base + g * CHUNK, CHUNK)], osem)

        load_idx(0, pidx0)
        fire_gather(pidx0, rows0, gsem0)

        def body(t, carry):
            g = 2 * t

            @pl.when(t > 0)
            def _():
                drain_out(rows1, osem1)

            load_idx(g + 1, pidx1)
            fire_gather(pidx1, rows1, gsem1)

            drain_gather(rows0, gsem0)
            fire_out(g, rows0, osem0)

            @pl.when(t < half_t - 1)
            def _():
                drain_out(rows0, osem0)
                load_idx(g + 2, pidx0)
                fire_gather(pidx0, rows0, gsem0)

            drain_gather(rows1, gsem1)
            fire_out(g + 1, rows1, osem1)
            return carry

        lax.fori_loop(0, half_t, body, 0)
        drain_out(rows0, osem0)
        drain_out(rows1, osem1)

    return k(pair_table, pidx)


def kernel(atomic_numbers, embedding_table):
    total = atomic_numbers.size
    idx = atomic_numbers.reshape(total // 2, 2).astype(jnp.int32)
    pidx = idx[:, 0] * VOCAB_ROWS + idx[:, 1]
    v = embedding_table.shape[0]
    pair_table = jnp.concatenate(
        [
            jnp.broadcast_to(embedding_table[:, None, :], (v, v, EMB)),
            jnp.broadcast_to(embedding_table[None, :, :], (v, v, EMB)),
        ],
        axis=-1,
    ).reshape(v * v, 2 * EMB)
    out = _sc_embedding_gather(pair_table, pidx, total)
    return out.reshape(atomic_numbers.shape + (EMB,))


# speedup vs baseline: 5.0752x; 1.3426x over previous
import functools

import jax
import jax.numpy as jnp
from jax import lax
from jax.experimental import pallas as pl
from jax.experimental.pallas import tpu as pltpu
from jax.experimental.pallas import tpu_sc as plsc

EMB = 64
VOCAB_ROWS = 129
CHUNK = 400
SUBS = ((0, 128), (128, 128), (256, 128), (384, 16))


@functools.partial(jax.jit, static_argnames=("total",))
def _sc_embedding_gather(pair_table, pidx, total):
    info = plsc.get_sparse_core_info()
    num_workers = info.num_cores * info.num_subcores
    pairs_total = total // 2
    per_worker = pairs_total // num_workers
    n_chunks = per_worker // CHUNK
    half_t = n_chunks // 2
    mesh = plsc.VectorSubcoreMesh(core_axis_name="c", subcore_axis_name="s")

    @functools.partial(
        pl.kernel,
        mesh=mesh,
        out_type=jax.ShapeDtypeStruct((pairs_total, 2 * EMB), jnp.float32),
        scratch_types=[
            pltpu.VMEM((CHUNK,), jnp.int32),
            pltpu.VMEM((CHUNK,), jnp.int32),
            pltpu.VMEM((CHUNK, 2 * EMB), jnp.float32),
            pltpu.VMEM((CHUNK, 2 * EMB), jnp.float32),
            pltpu.SemaphoreType.DMA,
            pltpu.SemaphoreType.DMA,
            pltpu.SemaphoreType.DMA,
            pltpu.SemaphoreType.DMA,
        ],
    )
    def k(pt_hbm, pidx_hbm, out_hbm, pidx0, pidx1, rows0, rows1,
          gsem0, gsem1, osem0, osem1):
        wid = lax.axis_index("s") * info.num_cores + lax.axis_index("c")
        base = wid * per_worker

        def fire_gather(pidx_v, rows_v, gsem):
            for off, sz in SUBS:
                pltpu.async_copy(
                    pt_hbm.at[pidx_v.at[pl.ds(off, sz)]],
                    rows_v.at[pl.ds(off, sz)],
                    gsem,
                )

        def drain_gather(rows_v, gsem):
            pltpu.make_async_copy(out_hbm.at[pl.ds(0, CHUNK)], rows_v, gsem).wait()

        def drain_out(rows_v, osem):
            pltpu.make_async_copy(rows_v, out_hbm.at[pl.ds(0, CHUNK)], osem).wait()

        def load_idx(g, pidx_v):
            pltpu.sync_copy(pidx_hbm.at[pl.ds(base + g * CHUNK, CHUNK)], pidx_v)

        def fire_out(g, rows_v, osem):
            pltpu.async_copy(rows_v, out_hbm.at[pl.ds(base + g * CHUNK, CHUNK)], osem)

        load_idx(0, pidx0)
        fire_gather(pidx0, rows0, gsem0)

        def body(t, carry):
            g = 2 * t

            @pl.when(t > 0)
            def _():
                drain_out(rows1, osem1)

            load_idx(g + 1, pidx1)
            fire_gather(pidx1, rows1, gsem1)

            drain_gather(rows0, gsem0)
            fire_out(g, rows0, osem0)

            @pl.when(t < half_t - 1)
            def _():
                drain_out(rows0, osem0)
                load_idx(g + 2, pidx0)
                fire_gather(pidx0, rows0, gsem0)

            drain_gather(rows1, gsem1)
            fire_out(g + 1, rows1, osem1)
            return carry

        lax.fori_loop(0, half_t, body, 0)
        drain_out(rows0, osem0)
        drain_out(rows1, osem1)

    return k(pair_table, pidx)


def kernel(atomic_numbers, embedding_table):
    total = atomic_numbers.size
    idx2d = atomic_numbers.astype(jnp.int32)
    pidx = (idx2d[:, 0::2] * VOCAB_ROWS + idx2d[:, 1::2]).reshape(total // 2)
    v = embedding_table.shape[0]
    left = jnp.pad(embedding_table, ((0, 0), (0, EMB)))
    right = jnp.pad(embedding_table, ((0, 0), (EMB, 0)))
    pair_table = (left[:, None, :] + right[None, :, :]).reshape(v * v, 2 * EMB)
    out = _sc_embedding_gather(pair_table, pidx, total)
    return out.reshape(atomic_numbers.shape + (EMB,))
